# baseline (speedup 1.0000x reference)
import functools

import jax
import jax.numpy as jnp
from jax import lax
from jax.experimental import pallas as pl
from jax.experimental.pallas import tpu as pltpu
from jax.experimental.pallas import tpu_sc as plsc

EPS = 1e-10
NEG_LOG_EPS = 23.025850929940457

VP = 1024
LANES = 16

NC = 2
NS = 16
NW = NC * NS


def _neglog_block(tab_ref, out_ref, *, v: int):
    nl = -jnp.log(tab_ref[...])
    rr = jax.lax.broadcasted_iota(jnp.int32, (VP, VP), 0)
    cc = jax.lax.broadcasted_iota(jnp.int32, (VP, VP), 1)
    sentinel = jnp.where(cc < v, NEG_LOG_EPS, 0.0)
    out_ref[...] = jnp.where(rr == v, sentinel, nl)


def _codes_block(x_ref, len_ref, codes_ref, carry_ref, *, tb: int, v: int):
    b = pl.program_id(0)
    t = pl.program_id(1)

    x = x_ref[0]

    maxv = jnp.max(x, axis=-1, keepdims=True)
    lane = jax.lax.broadcasted_iota(jnp.int32, (tb, v), 1)
    amax = jnp.min(jnp.where(x == maxv, lane, v),
                   axis=-1, keepdims=True)

    prev = jnp.where(t == 0, 1, carry_ref[0])
    rolled = jnp.roll(amax, 1, axis=0)
    row = jax.lax.broadcasted_iota(jnp.int32, (tb, 1), 0)
    codes = jnp.where(row == 0, prev, rolled)
    carry_ref[0] = amax[tb - 1, 0]

    tpos = t * tb + row
    codes = jnp.where(tpos >= len_ref[b], v, codes)
    codes_ref[...] = codes.reshape(1, 1, tb)


def _sc_dot(x_hbm, codes_hbm, neglog_hbm, out_hbm,
            idx_all, xb0, xb1, rb0, rb1, acc_io,
            sx0, sx1, sr0, sr1,
            *, v: int, tpw: int, ch: int):
    nch = tpw // ch
    wid = lax.axis_index("s") * NC + lax.axis_index("c")
    base = wid * tpw
    xw = ch * v

    zero16 = jnp.zeros((LANES,), jnp.float32)

    pltpu.sync_copy(codes_hbm.at[pl.ds(base, tpw)], idx_all)

    for xb in (xb0, xb1):
        xb[pl.ds(xw, LANES)] = zero16
        xb[pl.ds(xw + LANES, LANES)] = zero16

    def issue(g, xb, rb, sx, sr):
        cx = pltpu.make_async_copy(
            x_hbm.at[pl.ds((base + g * ch) * v, xw)], xb.at[pl.ds(0, xw)], sx)
        cr = pltpu.make_async_copy(
            neglog_hbm.at[idx_all.at[pl.ds(g * ch, ch)]], rb, sr)
        cx.start()
        cr.start()

    def wait(g, xb, rb, sx, sr):
        pltpu.make_async_copy(
            x_hbm.at[pl.ds((base + g * ch) * v, xw)],
            xb.at[pl.ds(0, xw)], sx).wait()
        pltpu.make_async_copy(
            neglog_hbm.at[idx_all.at[pl.ds(g * ch, ch)]], rb, sr).wait()

    tok16 = lax.iota(jnp.int32, LANES)
    bx = tok16 * v
    z16 = jnp.zeros((LANES,), jnp.int32)

    def compute(g, xb, rb, accs):
        def col_body(i, accs):
            a0, a1, a2, a3 = accs
            c = i * 4
            a0 += (plsc.load_gather(xb, [bx + (c + 0)])
                   * plsc.load_gather(rb, [tok16, z16 + (c + 0)]))
            a1 += (plsc.load_gather(xb, [bx + (c + 1)])
                   * plsc.load_gather(rb, [tok16, z16 + (c + 1)]))
            a2 += (plsc.load_gather(xb, [bx + (c + 2)])
                   * plsc.load_gather(rb, [tok16, z16 + (c + 2)]))
            a3 += (plsc.load_gather(xb, [bx + (c + 3)])
                   * plsc.load_gather(rb, [tok16, z16 + (c + 3)]))
            return (a0, a1, a2, a3)

        return lax.fori_loop(0, v // 4, col_body, accs)

    issue(0, xb0, rb0, sx0, sr0)
    issue(1, xb1, rb1, sx1, sr1)

    def pair_body(g2, accs):
        g = 2 * g2
        wait(g, xb0, rb0, sx0, sr0)
        accs = compute(g, xb0, rb0, accs)
        issue(g + 2, xb0, rb0, sx0, sr0)
        wait(g + 1, xb1, rb1, sx1, sr1)
        accs = compute(g + 1, xb1, rb1, accs)
        issue(g + 3, xb1, rb1, sx1, sr1)
        return accs

    accs = lax.fori_loop(0, nch // 2 - 1, pair_body,
                         (zero16, zero16, zero16, zero16))
    g = nch - 2
    wait(g, xb0, rb0, sx0, sr0)
    accs = compute(g, xb0, rb0, accs)
    wait(g + 1, xb1, rb1, sx1, sr1)
    accs = compute(g + 1, xb1, rb1, accs)

    acc = (accs[0] + accs[1]) + (accs[2] + accs[3])
    acc_io[...] = acc
    pltpu.sync_copy(acc_io, out_hbm.at[wid])


def kernel(enc_prob, enc_len, ngram_table):
    B, T, V = enc_prob.shape
    TB = 256
    NT = T // TB

    lens = enc_len.astype(jnp.int32)
    tab_pad = jnp.pad(ngram_table, ((0, VP - V), (0, VP - V)),
                      constant_values=1.0)

    neglog = pl.pallas_call(
        functools.partial(_neglog_block, v=V),
        out_shape=jax.ShapeDtypeStruct((VP, VP), jnp.float32),
    )(tab_pad)

    codes3 = pl.pallas_call(
        functools.partial(_codes_block, tb=TB, v=V),
        grid=(B, NT),
        in_specs=[
            pl.BlockSpec((1, TB, V), lambda b, t: (b, t, 0)),
            pl.BlockSpec(memory_space=pltpu.SMEM),
        ],
        out_specs=pl.BlockSpec((1, 1, TB), lambda b, t: (b * NT + t, 0, 0)),
        out_shape=jax.ShapeDtypeStruct((B * NT, 1, TB), jnp.int32),
        scratch_shapes=[pltpu.SMEM((1,), jnp.int32)],
    )(enc_prob, lens)

    codes = codes3.reshape(B * T)
    x_flat = enc_prob.reshape(B * T * V)

    TPW = (B * T) // NW
    CH = 16

    mesh = plsc.VectorSubcoreMesh(core_axis_name="c", subcore_axis_name="s")
    parts = pl.kernel(
        functools.partial(_sc_dot, v=V, tpw=TPW, ch=CH),
        out_type=jax.ShapeDtypeStruct((NW, LANES), jnp.float32),
        mesh=mesh,
        compiler_params=pltpu.CompilerParams(use_tc_tiling_on_sc=False,
                                             needs_layout_passes=False),
        scratch_types=[
            pltpu.VMEM((TPW,), jnp.int32),
            pltpu.VMEM((CH * V + 2 * LANES,), jnp.float32),
            pltpu.VMEM((CH * V + 2 * LANES,), jnp.float32),
            pltpu.VMEM((CH, VP), jnp.float32),
            pltpu.VMEM((CH, VP), jnp.float32),
            pltpu.VMEM((LANES,), jnp.float32),
            pltpu.SemaphoreType.DMA,
            pltpu.SemaphoreType.DMA,
            pltpu.SemaphoreType.DMA,
            pltpu.SemaphoreType.DMA,
        ],
    )(x_flat, codes, neglog)

    w = 1.0 / (enc_len.astype(jnp.float32) * B)
    return jnp.sum(parts.reshape(B, 2, LANES) * w[:, None, None])


# speedup vs baseline: 4.5377x; 4.5377x over previous
import functools

import jax
import jax.numpy as jnp
from jax.experimental import pallas as pl
from jax.experimental.pallas import tpu as pltpu

EPS = 1e-10
NEG_LOG_EPS = 23.025850929940457


def _kld_block(x_ref, len_ref, w_ref, tab_ref, out_ref, neglog_ref, carry_ref,
               *, tb: int, v: int, nt: int):
    b = pl.program_id(0)
    t = pl.program_id(1)

    @pl.when(jnp.logical_and(b == 0, t == 0))
    def _init_table():
        neglog_ref[...] = (-jnp.log(tab_ref[...])).astype(jnp.bfloat16)

    @pl.when(jnp.logical_and(b == 0, t == 0))
    def _init_out():
        out_ref[...] = jnp.zeros((1, 1), jnp.float32)

    x = x_ref[0]

    maxv = jnp.max(x, axis=-1, keepdims=True)
    lane = jax.lax.broadcasted_iota(jnp.int32, (tb, v), 1)
    amax = jnp.min(jnp.where(x == maxv, lane, v),
                   axis=-1, keepdims=True)

    prev = jnp.where(t == 0, 1, carry_ref[0])
    rolled = jnp.roll(amax, 1, axis=0)
    row = jax.lax.broadcasted_iota(jnp.int32, (tb, 1), 0)
    codes = jnp.where(row == 0, prev, rolled)
    carry_ref[0] = amax[tb - 1, 0]

    onehot = (codes == jax.lax.broadcasted_iota(jnp.int32, (tb, v), 1))
    oh = onehot.astype(jnp.bfloat16)
    dims = (((1,), (0,)), ((), ()))
    g = jax.lax.dot_general(oh, neglog_ref[...], dims,
                            preferred_element_type=jnp.float32)

    dots = jnp.sum(x * g, axis=-1, keepdims=True)
    rowsum = jnp.sum(x, axis=-1, keepdims=True)

    tpos = t * tb + row
    masked = tpos >= len_ref[b]
    val = jnp.where(masked, NEG_LOG_EPS * rowsum, dots)
    out_ref[...] += jnp.full((1, 1), jnp.sum(val) * w_ref[b], jnp.float32)


def kernel(enc_prob, enc_len, ngram_table):
    B, T, V = enc_prob.shape
    TB = 512
    NT = T // TB

    lens = enc_len.astype(jnp.int32)
    w = 1.0 / (enc_len.astype(jnp.float32) * B)

    out = pl.pallas_call(
        functools.partial(_kld_block, tb=TB, v=V, nt=NT),
        grid=(B, NT),
        in_specs=[
            pl.BlockSpec((1, TB, V), lambda b, t: (b, t, 0)),
            pl.BlockSpec(memory_space=pltpu.SMEM),
            pl.BlockSpec(memory_space=pltpu.SMEM),
            pl.BlockSpec((V, V), lambda b, t: (0, 0)),
        ],
        out_specs=pl.BlockSpec((1, 1), lambda b, t: (0, 0)),
        out_shape=jax.ShapeDtypeStruct((1, 1), jnp.float32),
        scratch_shapes=[
            pltpu.VMEM((V, V), jnp.bfloat16),
            pltpu.SMEM((1,), jnp.int32),
        ],
    )(enc_prob, lens, w, ngram_table)
    return out[0, 0]


# speedup vs baseline: 4.8584x; 1.0707x over previous
import functools

import jax
import jax.numpy as jnp
from jax.experimental import pallas as pl
from jax.experimental.pallas import tpu as pltpu

EPS = 1e-10
NEG_LOG_EPS = 23.025850929940457


def _kld_block(x_ref, len_ref, w_ref, tab_ref, out_ref, neglog_ref, carry_ref,
               *, tb: int, v: int, nt: int):
    b = pl.program_id(0)
    t = pl.program_id(1)

    @pl.when(jnp.logical_and(b == 0, t == 0))
    def _init_table():
        neglog_ref[...] = (-jnp.log(tab_ref[...])).astype(jnp.bfloat16)

    @pl.when(jnp.logical_and(b == 0, t == 0))
    def _init_out():
        out_ref[...] = jnp.zeros((1, 1), jnp.float32)

    x = x_ref[0]

    maxv = jnp.max(x, axis=-1, keepdims=True)
    lane = jax.lax.broadcasted_iota(jnp.int32, (tb, v), 1)
    amax = jnp.min(jnp.where(x == maxv, lane, v),
                   axis=-1, keepdims=True)

    prev = jnp.where(t == 0, 1, carry_ref[0])
    rolled = jnp.roll(amax, 1, axis=0)
    row = jax.lax.broadcasted_iota(jnp.int32, (tb, 1), 0)
    codes = jnp.where(row == 0, prev, rolled)
    carry_ref[0] = amax[tb - 1, 0]

    onehot = (codes == jax.lax.broadcasted_iota(jnp.int32, (tb, v), 1))
    oh = onehot.astype(jnp.bfloat16)
    dims = (((1,), (0,)), ((), ()))
    g = jax.lax.dot_general(oh, neglog_ref[...], dims,
                            preferred_element_type=jnp.float32)

    dots = jnp.sum(x * g, axis=-1, keepdims=True)
    rowsum = jnp.sum(x, axis=-1, keepdims=True)

    tpos = t * tb + row
    masked = tpos >= len_ref[b]
    val = jnp.where(masked, NEG_LOG_EPS * rowsum, dots)
    out_ref[...] += jnp.full((1, 1), jnp.sum(val) * w_ref[b], jnp.float32)


def kernel(enc_prob, enc_len, ngram_table):
    B, T, V = enc_prob.shape
    TB = 1024
    NT = T // TB

    lens = enc_len.astype(jnp.int32)
    w = 1.0 / (enc_len.astype(jnp.float32) * B)

    out = pl.pallas_call(
        functools.partial(_kld_block, tb=TB, v=V, nt=NT),
        grid=(B, NT),
        in_specs=[
            pl.BlockSpec((1, TB, V), lambda b, t: (b, t, 0)),
            pl.BlockSpec(memory_space=pltpu.SMEM),
            pl.BlockSpec(memory_space=pltpu.SMEM),
            pl.BlockSpec((V, V), lambda b, t: (0, 0)),
        ],
        out_specs=pl.BlockSpec((1, 1), lambda b, t: (0, 0)),
        out_shape=jax.ShapeDtypeStruct((1, 1), jnp.float32),
        scratch_shapes=[
            pltpu.VMEM((V, V), jnp.bfloat16),
            pltpu.SMEM((1,), jnp.int32),
        ],
    )(enc_prob, lens, w, ngram_table)
    return out[0, 0]


# speedup vs baseline: 4.9317x; 1.0151x over previous
import functools

import jax
import jax.numpy as jnp
from jax.experimental import pallas as pl
from jax.experimental.pallas import tpu as pltpu

EPS = 1e-10
NEG_LOG_EPS = 23.025850929940457


def _kld_block(x_ref, len_ref, w_ref, tab_ref, out_ref, neglog_ref, carry_ref,
               *, tb: int, v: int, nt: int):
    b = pl.program_id(0)
    t = pl.program_id(1)

    @pl.when(jnp.logical_and(b == 0, t == 0))
    def _init_table():
        neglog_ref[...] = (-jnp.log(tab_ref[...])).astype(jnp.bfloat16)

    @pl.when(jnp.logical_and(b == 0, t == 0))
    def _init_out():
        out_ref[...] = jnp.zeros((1, 1), jnp.float32)

    x = x_ref[0]

    maxv = jnp.max(x, axis=-1, keepdims=True)
    lane = jax.lax.broadcasted_iota(jnp.int32, (tb, v), 1)
    amax = jnp.min(jnp.where(x == maxv, lane, v),
                   axis=-1, keepdims=True)

    prev = jnp.where(t == 0, 1, carry_ref[0])
    rolled = jnp.roll(amax, 1, axis=0)
    row = jax.lax.broadcasted_iota(jnp.int32, (tb, 1), 0)
    codes = jnp.where(row == 0, prev, rolled)
    carry_ref[0] = amax[tb - 1, 0]

    onehot = (codes == jax.lax.broadcasted_iota(jnp.int32, (tb, v), 1))
    oh = onehot.astype(jnp.bfloat16)
    dims = (((1,), (0,)), ((), ()))
    g = jax.lax.dot_general(oh, neglog_ref[...], dims,
                            preferred_element_type=jnp.float32)

    dots = jnp.sum(x * g, axis=-1, keepdims=True)
    rowsum = jnp.sum(x, axis=-1, keepdims=True)

    tpos = t * tb + row
    masked = tpos >= len_ref[b]
    val = jnp.where(masked, NEG_LOG_EPS * rowsum, dots)
    out_ref[...] += jnp.full((1, 1), jnp.sum(val) * w_ref[b], jnp.float32)


def kernel(enc_prob, enc_len, ngram_table):
    B, T, V = enc_prob.shape
    TB = 2048
    NT = T // TB

    lens = enc_len.astype(jnp.int32)
    w = 1.0 / (enc_len.astype(jnp.float32) * B)

    out = pl.pallas_call(
        functools.partial(_kld_block, tb=TB, v=V, nt=NT),
        grid=(B, NT),
        in_specs=[
            pl.BlockSpec((1, TB, V), lambda b, t: (b, t, 0)),
            pl.BlockSpec(memory_space=pltpu.SMEM),
            pl.BlockSpec(memory_space=pltpu.SMEM),
            pl.BlockSpec((V, V), lambda b, t: (0, 0)),
        ],
        out_specs=pl.BlockSpec((1, 1), lambda b, t: (0, 0)),
        out_shape=jax.ShapeDtypeStruct((1, 1), jnp.float32),
        scratch_shapes=[
            pltpu.VMEM((V, V), jnp.bfloat16),
            pltpu.SMEM((1,), jnp.int32),
        ],
    )(enc_prob, lens, w, ngram_table)
    return out[0, 0]
